# Initial kernel scaffold; baseline (speedup 1.0000x reference)
import jax
import jax.numpy as jnp
from jax.experimental import pallas as pl


def kernel(x, edge_index, topo, Wt1, bt1, Wt2, bt2, W1, b1, W2, b2, Wout, bout):
    raise NotImplementedError("write your pallas kernel here")


# speedup vs baseline: 8.0822x; 8.0822x over previous
import functools

import jax
import jax.numpy as jnp
from jax import lax
from jax.experimental import pallas as pl
from jax.experimental.pallas import tpu as pltpu
from jax.experimental.pallas import tpu_sc as plsc

_NC = 2
_NS = 16
_L = 16
_CH = 128


_DW = 128


def _make_sc_degree(n, n_acc, e_pad):
    epw = e_pad // (_NC * _NS)
    nchunks = epw // _CH
    rpt = n_acc // _NS

    def body(dst_hbm, ones_hbm, zeros_hbm, out_hbm, ones_v, idx_v, acc_sh):
        c = lax.axis_index("c")
        s = lax.axis_index("s")
        wid = s * _NC + c

        pltpu.sync_copy(ones_hbm, ones_v)
        pltpu.sync_copy(zeros_hbm, acc_sh.at[pl.ds(s * rpt, rpt)])
        plsc.subcore_barrier()

        base = wid * epw

        @pl.loop(0, nchunks)
        def _chunk(i):
            pltpu.sync_copy(dst_hbm.at[pl.ds(base + i * _CH, _CH)], idx_v)
            pltpu.sync_copy(ones_v, acc_sh.at[idx_v], add=True)

        plsc.subcore_barrier()
        cpt = (n // _NS) // 8 * 8
        tail = n - _NS * cpt
        pltpu.sync_copy(acc_sh.at[pl.ds(s * cpt, cpt)],
                        out_hbm.at[pl.ds(c * n + s * cpt, cpt)])
        if tail:
            @pl.when(s == _NS - 1)
            def _tail():
                pltpu.sync_copy(acc_sh.at[pl.ds(n - tail, tail)],
                                out_hbm.at[pl.ds(c * n + n - tail, tail)])

    mesh = plsc.VectorSubcoreMesh(core_axis_name="c", subcore_axis_name="s")
    return pl.kernel(
        body,
        out_type=jax.ShapeDtypeStruct((_NC * n, _DW), jnp.float32),
        mesh=mesh,
        scratch_types=[
            pltpu.VMEM((_CH, _DW), jnp.float32),
            pltpu.VMEM((_CH,), jnp.int32),
            pltpu.VMEM_SHARED((n_acc, _DW), jnp.float32),
        ],
    )


def _make_sc_agg(n, n_acc, e_pad, d):
    epw = e_pad // _NS
    nchunks = epw // _CH
    rpt = n_acc // _NS

    def body(table_hbm, srcs_hbm, dst_hbm, zeros_hbm, out_hbm,
             sidx_v, didx_v, rows_v, acc_sh, sem):
        c = lax.axis_index("c")
        s = lax.axis_index("s")

        pltpu.sync_copy(zeros_hbm, acc_sh.at[pl.ds(s * rpt, rpt)])
        plsc.subcore_barrier()

        base = s * epw

        @pl.loop(0, nchunks)
        def _chunk(i):
            eb = base + i * _CH
            pltpu.sync_copy(srcs_hbm.at[pl.ds(c * e_pad + eb, _CH)], sidx_v)
            pltpu.sync_copy(dst_hbm.at[pl.ds(eb, _CH)], didx_v)
            pltpu.async_copy(table_hbm.at[sidx_v], rows_v, sem).wait()
            pltpu.sync_copy(rows_v, acc_sh.at[didx_v], add=True)

        plsc.subcore_barrier()
        cpt = (n // _NS) // 8 * 8
        tail = n - _NS * cpt
        pltpu.sync_copy(acc_sh.at[pl.ds(s * cpt, cpt)],
                        out_hbm.at[pl.ds(c * n + s * cpt, cpt)])
        if tail:
            @pl.when(s == _NS - 1)
            def _tail():
                pltpu.sync_copy(acc_sh.at[pl.ds(n - tail, tail)],
                                out_hbm.at[pl.ds(c * n + n - tail, tail)])

    mesh = plsc.VectorSubcoreMesh(core_axis_name="c", subcore_axis_name="s")
    return pl.kernel(
        body,
        out_type=jax.ShapeDtypeStruct((_NC * n, d), jnp.float32),
        mesh=mesh,
        scratch_types=[
            pltpu.VMEM((_CH,), jnp.int32),
            pltpu.VMEM((_CH,), jnp.int32),
            pltpu.VMEM((_CH, d), jnp.float32),
            pltpu.VMEM_SHARED((n_acc, d), jnp.float32),
            pltpu.SemaphoreType.DMA,
        ],
    )


def _tc1_body(x_r, topo_r, d0_r, d1_r, wt1_r, bt1_r, wt2_r, bt2_r, w1_r,
              out_r, dinv_r):
    tz = jnp.maximum(
        jnp.dot(topo_r[...], wt1_r[...], preferred_element_type=jnp.float32)
        + bt1_r[...], 0.0)
    tz = jnp.maximum(
        jnp.dot(tz, wt2_r[...], preferred_element_type=jnp.float32)
        + bt2_r[...], 0.0)
    h = jnp.concatenate([x_r[...], tz], axis=1)
    hw = jnp.dot(h, w1_r[...], preferred_element_type=jnp.float32)
    deg = d0_r[:, 0:1] + d1_r[:, 0:1] + 1.0
    dinv = lax.rsqrt(deg)
    hwp = hw * dinv
    half = hw.shape[1] // 2
    out_r[0] = hwp[:, :half]
    out_r[1] = hwp[:, half:]
    dinv_r[...] = dinv


def _tc_mid_body(agg_r, hwp_r, dinv_r, b_r, w_r, out_r):
    sfull = jnp.concatenate(
        [agg_r[0] + hwp_r[0], agg_r[1] + hwp_r[1]], axis=1)
    hcur = jnp.maximum(dinv_r[...] * sfull + b_r[...], 0.0)
    hw = jnp.dot(hcur, w_r[...], preferred_element_type=jnp.float32)
    hwp = hw * dinv_r[...]
    half = hw.shape[1] // 2
    out_r[0] = hwp[:, :half]
    out_r[1] = hwp[:, half:]


def _tc_out_body(agg_r, hwp_r, dinv_r, b_r, wout_r, bout_r, out_r):
    sfull = jnp.concatenate(
        [agg_r[0] + hwp_r[0], agg_r[1] + hwp_r[1]], axis=1)
    hcur = jnp.maximum(dinv_r[...] * sfull + b_r[...], 0.0)
    o = jnp.dot(hcur, wout_r[...], preferred_element_type=jnp.float32)
    o = o + bout_r[...]
    out_r[...] = 1.0 / (1.0 + jnp.exp(-o))


def _full2(a):
    return pl.BlockSpec(a.shape, lambda i: (0, 0))


def kernel(x, edge_index, topo, Wt1, bt1, Wt2, bt2, W1, b1, W2, b2,
           Wout, bout):
    n, d_in = x.shape
    hid = W1.shape[1]
    half = hid // 2
    src = edge_index[0]
    dst = edge_index[1]
    e = src.shape[0]

    quant = _NC * _NS * _CH
    e_pad = -(-e // quant) * quant
    pad = e_pad - e
    n_acc = -(-(n + 1) // (_NS * 8)) * (_NS * 8)

    src_p = jnp.concatenate([src, jnp.zeros((pad,), src.dtype)])
    dst_p = jnp.concatenate([dst, jnp.full((pad,), n, dst.dtype)])
    srcs2 = jnp.concatenate([src_p, src_p + n])

    rpt_deg = n_acc // _NS
    deg_out = _make_sc_degree(n, n_acc, e_pad)(
        dst_p, jnp.ones((_CH, _DW), jnp.float32),
        jnp.zeros((rpt_deg, _DW), jnp.float32))
    d0 = deg_out[0:n, 0:_L]
    d1 = deg_out[n:2 * n, 0:_L]

    B = 1000
    grid = (n // B,)
    row = lambda shp: pl.BlockSpec(shp, lambda i: (i, 0))
    row3 = lambda shp: pl.BlockSpec(shp, lambda i: (0, i, 0))
    table1, dinv = pl.pallas_call(
        _tc1_body,
        grid=grid,
        in_specs=[
            row((B, d_in)), row((B, topo.shape[1])),
            row((B, _L)), row((B, _L)),
            _full2(Wt1), _full2(bt1.reshape(1, -1)),
            _full2(Wt2), _full2(bt2.reshape(1, -1)),
            _full2(W1),
        ],
        out_specs=[row3((2, B, half)), row((B, 1))],
        out_shape=[
            jax.ShapeDtypeStruct((2, n, half), jnp.float32),
            jax.ShapeDtypeStruct((n, 1), jnp.float32),
        ],
    )(x, topo, d0, d1, Wt1, bt1.reshape(1, -1), Wt2, bt2.reshape(1, -1), W1)

    agg_call = _make_sc_agg(n, n_acc, e_pad, half)
    zeros_agg = jnp.zeros((n_acc // _NS, half), jnp.float32)

    agg1 = agg_call(table1.reshape(2 * n, half), srcs2, dst_p, zeros_agg)
    agg1 = agg1.reshape(2, n, half)
    table2 = pl.pallas_call(
        _tc_mid_body,
        grid=grid,
        in_specs=[
            row3((2, B, half)), row3((2, B, half)), row((B, 1)),
            _full2(b1.reshape(1, -1)), _full2(W2),
        ],
        out_specs=row3((2, B, half)),
        out_shape=jax.ShapeDtypeStruct((2, n, half), jnp.float32),
    )(agg1, table1, dinv, b1.reshape(1, -1), W2)

    agg2 = agg_call(table2.reshape(2 * n, half), srcs2, dst_p, zeros_agg)
    agg2 = agg2.reshape(2, n, half)
    out = pl.pallas_call(
        _tc_out_body,
        grid=grid,
        in_specs=[
            row3((2, B, half)), row3((2, B, half)), row((B, 1)),
            _full2(b2.reshape(1, -1)), _full2(Wout),
            _full2(bout.reshape(1, -1)),
        ],
        out_specs=row((B, 1)),
        out_shape=jax.ShapeDtypeStruct((n, 1), jnp.float32),
    )(agg2, table2, dinv, b2.reshape(1, -1), Wout, bout.reshape(1, -1))
    return out
